# baseline (speedup 1.0000x reference)
import functools

import jax
import jax.numpy as jnp
import numpy as np
from jax import lax
from jax.experimental import pallas as pl
from jax.experimental.pallas import tpu as pltpu
from jax.experimental.pallas import tpu_sc as plsc

_B, _S, _VOCAB, _D = 4096, 200, 1000000, 64
_SCALE = float(np.sqrt(_D))
_NC, _NS, _L = 2, 16, 16
_NW = _NC * _NS
_CB = _B // _NW
_CH = 128
_NFULL = _VOCAB // _CH
_TAIL = _VOCAB - _NFULL * _CH


def _positional_encoding_np(max_len, d_model):
    pos = np.arange(max_len, dtype=np.float32)[:, None]
    div = np.exp(np.arange(0, d_model, 2, dtype=np.float32)
                 * (-np.log(10000.0) / d_model))
    pe = np.zeros((max_len, d_model), dtype=np.float32)
    pe[:, 0::2] = np.sin(pos * div)
    pe[:, 1::2] = np.cos(pos * div)
    return pe


_PE_FLAT = _positional_encoding_np(_S, _D).reshape(-1)


def _sc_body(xt_hbm, embt_hbm, pe_hbm, tail_hbm, outp_hbm, scr_hbm,
             pb0, pb1, pb2, pb3, ib0, ib1, ib2, ib3,
             g0, g1, g2, o0, o1,
             tis0, tis1, tos0, tos1, ps0, ps1, ps2, ps3,
             is0, is1, is2, is3, gs0, gs1, gs2, os0, os1):
    pbuf = (pb0, pb1, pb2, pb3)
    psem = (ps0, ps1, ps2, ps3)
    ibuf = (ib0, ib1, ib2, ib3)
    isem = (is0, is1, is2, is3)
    g = (g0, g1, g2)
    gsem = (gs0, gs1, gs2)
    o = (o0, o1)
    osem = (os0, os1)
    ta_in = (o0, o1)
    ta_out = (g0, g1)
    tisem = (tis0, tis1)
    tosem = (tos0, tos1)

    cid = lax.axis_index("c")
    sid = lax.axis_index("s")
    wid = sid * _NC + cid
    lane = lax.iota(jnp.int32, _L)
    skew = [lax.rem(lane + k, _L) for k in range(_L)]

    def transpose_16x16(src, dst, db, cb):
        rowv = lax.broadcast(db * _L, (_L,)) + lane
        cbase = lax.broadcast(cb * _L, (_L,))
        for k in range(_L):
            cv = cbase + skew[k]
            v = plsc.load_gather(src, [rowv, cv])
            plsc.store_scatter(dst, [cv, rowv], v)

    nfull_w = _NFULL // _NS + jnp.where(sid < (_NFULL % _NS), 1, 0)

    def a_in_desc(p, i):
        c0 = (sid + _NS * i) * _CH
        return pltpu.make_async_copy(
            embt_hbm.at[:, pl.ds(c0, _CH)], ta_in[p], tisem[p])

    def a_out_desc(p, i):
        c0 = (sid + _NS * i) * _CH
        return pltpu.make_async_copy(
            ta_out[p], scr_hbm.at[pl.ds(c0, _CH), :], tosem[p])

    @pl.when(0 < nfull_w)
    def _():
        a_in_desc(0, 0).start()

    @pl.when(1 < nfull_w)
    def _():
        a_in_desc(1, 1).start()

    def a_chunk(ii, carry):
        p = lax.rem(ii, 2)
        for pp in range(2):
            @pl.when((p == pp) & (ii < nfull_w))
            def _():
                a_in_desc(pp, ii).wait()

                @pl.when(ii >= 2)
                def _():
                    a_out_desc(pp, ii - 2).wait()

                for db in range(_D // _L):
                    def cb_body(cb, c2):
                        transpose_16x16(ta_in[pp], ta_out[pp], db, cb)
                        return c2
                    lax.fori_loop(0, _CH // _L, cb_body, 0)
                a_out_desc(pp, ii).start()

                @pl.when(ii + 2 < nfull_w)
                def _():
                    a_in_desc(pp, ii + 2).start()
        return carry

    lax.fori_loop(0, _NFULL // _NS + 1, a_chunk, 0)
    for p in range(2):
        a_out_desc(p, p).wait()

    pltpu.sync_copy(tail_hbm, g0.at[pl.ds(0, _TAIL), :])
    pltpu.sync_copy(g0.at[pl.ds(0, _TAIL), :],
                    scr_hbm.at[pl.ds(_NFULL * _CH, _TAIL), :])

    plsc.subcore_barrier()

    b0 = wid * _CB

    def b_idx_desc(s, jb):
        return pltpu.make_async_copy(
            xt_hbm.at[s, pl.ds(b0, _CB)], ibuf[jb % 4], isem[jb % 4])

    def b_pe_desc(s, jb):
        return pltpu.make_async_copy(
            pe_hbm.at[pl.ds(s * _D, _D)], pbuf[jb % 4], psem[jb % 4])

    def b_gather_desc(jb):
        return pltpu.make_async_copy(
            scr_hbm.at[ibuf[jb % 4]], g[jb % 3], gsem[jb % 3])

    def b_out_desc(s, ob):
        return pltpu.make_async_copy(
            o[ob], outp_hbm.at[s, :, pl.ds(b0, _CB)], osem[ob])

    def b_compute(s, gb, ib, ob):
        def db_body(db, c1):
            dbase = lax.broadcast(db * _L, (_L,))
            dvs = [dbase + skew[k] for k in range(_L)]
            pes = [plsc.load_gather(pbuf[ib], [dvs[k]]) for k in range(_L)]

            def bb_body(bb, c2):
                bv = lax.broadcast(bb * _L, (_L,)) + lane
                for k in range(_L):
                    v = plsc.load_gather(g[gb], [bv, dvs[k]])
                    plsc.store_scatter(o[ob], [dvs[k], bv],
                                       v * _SCALE + pes[k])
                return c2

            lax.fori_loop(0, _CB // _L, bb_body, 0)
            return c1

        lax.fori_loop(0, _D // _L, db_body, 0)

    for s0 in range(4):
        b_idx_desc(s0, s0).start()
        b_pe_desc(s0, s0).start()
    for s0 in range(2):
        b_idx_desc(s0, s0).wait()
        b_gather_desc(s0).start()

    def b_phase(s, j):
        ib, gb, ob = j % 4, j % 3, j % 2

        @pl.when(s < _S)
        def _():
            b_gather_desc(j).wait()
            b_pe_desc(s, j).wait()

            @pl.when(s >= 2)
            def _():
                b_out_desc(s - 2, ob).wait()

            b_compute(s, gb, ib, ob)
            b_out_desc(s, ob).start()

            @pl.when(s + 2 < _S)
            def _():
                pltpu.make_async_copy(
                    xt_hbm.at[s + 2, pl.ds(b0, _CB)],
                    ibuf[(j + 2) % 4], isem[(j + 2) % 4]).wait()
                b_gather_desc(j + 2).start()

            @pl.when(s + 4 < _S)
            def _():
                b_idx_desc(s + 4, j).start()
                b_pe_desc(s + 4, j).start()

    def b_group(gg, carry):
        for j in range(12):
            b_phase(gg * 12 + j, j)
        return carry

    lax.fori_loop(0, (_S + 11) // 12, b_group, 0)
    b_out_desc(_S - 2, 0).wait()
    b_out_desc(_S - 1, 1).wait()


@jax.jit
def _run(xt, embt, pe, tail):
    mesh = plsc.VectorSubcoreMesh(core_axis_name="c", subcore_axis_name="s")
    f = functools.partial(
        pl.kernel,
        mesh=mesh,
        out_type=(
            jax.ShapeDtypeStruct((_S, _D, _B), jnp.float32),
            jax.ShapeDtypeStruct((_VOCAB, 128), jnp.float32),
        ),
        scratch_types=[
            pltpu.VMEM((_D,), jnp.float32),
            pltpu.VMEM((_D,), jnp.float32),
            pltpu.VMEM((_D,), jnp.float32),
            pltpu.VMEM((_D,), jnp.float32),
            pltpu.VMEM((_CB,), jnp.int32),
            pltpu.VMEM((_CB,), jnp.int32),
            pltpu.VMEM((_CB,), jnp.int32),
            pltpu.VMEM((_CB,), jnp.int32),
            pltpu.VMEM((_CB, 128), jnp.float32),
            pltpu.VMEM((_CB, 128), jnp.float32),
            pltpu.VMEM((_CB, 128), jnp.float32),
            pltpu.VMEM((_D, _CB), jnp.float32),
            pltpu.VMEM((_D, _CB), jnp.float32),
            pltpu.SemaphoreType.DMA,
            pltpu.SemaphoreType.DMA,
            pltpu.SemaphoreType.DMA,
            pltpu.SemaphoreType.DMA,
            pltpu.SemaphoreType.DMA,
            pltpu.SemaphoreType.DMA,
            pltpu.SemaphoreType.DMA,
            pltpu.SemaphoreType.DMA,
            pltpu.SemaphoreType.DMA,
            pltpu.SemaphoreType.DMA,
            pltpu.SemaphoreType.DMA,
            pltpu.SemaphoreType.DMA,
            pltpu.SemaphoreType.DMA,
            pltpu.SemaphoreType.DMA,
            pltpu.SemaphoreType.DMA,
            pltpu.SemaphoreType.DMA,
            pltpu.SemaphoreType.DMA,
        ],
        compiler_params=pltpu.CompilerParams(
            use_tc_tiling_on_sc=True, needs_layout_passes=False),
    )(_sc_body)
    outp, _ = f(xt, embt, pe, tail)
    return jnp.transpose(outp, (2, 0, 1))


def kernel(x, emb):
    xt = jnp.transpose(x.astype(jnp.int32))
    embt = jnp.transpose(emb)
    tail = jnp.pad(emb[_NFULL * _CH:, :], ((0, 0), (0, 128 - _D)))
    return _run(xt, embt, jnp.asarray(_PE_FLAT), tail)
